# Initial kernel scaffold; baseline (speedup 1.0000x reference)
import jax
import jax.numpy as jnp
from jax.experimental import pallas as pl


def kernel(x, y, embed_weight, anc_idx):
    raise NotImplementedError("write your pallas kernel here")


# speedup vs baseline: 1.0467x; 1.0467x over previous
import functools

import jax
import jax.numpy as jnp
from jax import lax
from jax.experimental import pallas as pl
from jax.experimental.pallas import tpu as pltpu
from jax.experimental.pallas import tpu_sc as plsc

A = 100000
D = 200
C = 16384
K = 16
B = 1024

NC = 2
NS = 16
NW = NC * NS
CW = C // NW
NCH = 8
RCH = NCH * K
NCHUNK = CW // NCH

OFFS = tuple(range(0, 192, 16)) + (184,)


def _sc_gather_sum(table, idx_flat):
    mesh = plsc.VectorSubcoreMesh(core_axis_name="c", subcore_axis_name="s")

    @functools.partial(
        pl.kernel,
        out_type=jax.ShapeDtypeStruct((C, D), jnp.float32),
        mesh=mesh,
        scratch_types=[
            pltpu.VMEM((CW * K,), jnp.int32),
            pltpu.VMEM((RCH, D), jnp.float32),
            pltpu.VMEM((NCH, D), jnp.float32),
            pltpu.SemaphoreType.DMA,
        ],
        compiler_params=pltpu.CompilerParams(use_tc_tiling_on_sc=False),
    )
    def body(table_hbm, idx_hbm, out_hbm, idx_v, stg_v, acc_v, sem):
        cid = lax.axis_index("c")
        sid = lax.axis_index("s")
        wid = sid * NC + cid
        base = wid * CW
        pltpu.sync_copy(idx_hbm.at[pl.ds(base * K, CW * K)], idx_v)

        @pl.loop(0, NCHUNK)
        def _chunk(ch):
            roff = pl.multiple_of(ch * RCH, 8)
            pltpu.async_copy(
                table_hbm.at[idx_v.at[pl.ds(roff, RCH)]], stg_v, sem
            ).wait()

            @pl.loop(0, NCH)
            def _concept(i):
                row0 = i * K

                def kbody(k, accs):
                    return tuple(
                        a + stg_v[row0 + k, pl.ds(o, 16)]
                        for a, o in zip(accs, OFFS)
                    )

                accs = tuple(stg_v[row0, pl.ds(o, 16)] for o in OFFS)
                accs = lax.fori_loop(1, K, kbody, accs)
                for a, o in zip(accs, OFFS):
                    acc_v[i, pl.ds(o, 16)] = a

            pltpu.sync_copy(acc_v, out_hbm.at[pl.ds(base + ch * NCH, NCH)])

    return body(table, idx_flat)


def _tc_head(x2, local_h):
    bt = 128

    def body(x_ref, h_ref, o_ref):
        logits = lax.dot_general(
            x_ref[...], h_ref[...], (((1,), (1,)), ((), ())),
            preferred_element_type=jnp.float32,
            precision=lax.Precision.DEFAULT,
        )
        m = jnp.max(logits, axis=1, keepdims=True)
        e = jnp.exp(logits - m)
        o_ref[...] = e / jnp.sum(e, axis=1, keepdims=True)

    return pl.pallas_call(
        body,
        grid=(B // bt,),
        in_specs=[
            pl.BlockSpec((bt, D), lambda i: (i, 0)),
            pl.BlockSpec((C, D), lambda i: (0, 0)),
        ],
        out_specs=pl.BlockSpec((bt, C), lambda i: (i, 0)),
        out_shape=jax.ShapeDtypeStruct((B, C), jnp.float32),
    )(x2, local_h)


def kernel(x, y, embed_weight, anc_idx):
    idx_flat = anc_idx.reshape(-1).astype(jnp.int32)
    local_h = _sc_gather_sum(embed_weight, idx_flat)
    return _tc_head(x.reshape(B, D), local_h)


# speedup vs baseline: 1.1403x; 1.0895x over previous
import functools

import jax
import jax.numpy as jnp
from jax import lax
from jax.experimental import pallas as pl
from jax.experimental.pallas import tpu as pltpu
from jax.experimental.pallas import tpu_sc as plsc

A = 100000
D = 200
DP = 256
C = 16384
K = 16
B = 1024

NC = 2
NS = 16
NW = NC * NS
CW = C // NW
NCH = 8
RCH = NCH * K
NCHUNK = CW // NCH

OFFS = tuple(range(0, DP, 16))


def _sc_gather_sum(table, idx_flat):
    mesh = plsc.VectorSubcoreMesh(core_axis_name="c", subcore_axis_name="s")

    @functools.partial(
        pl.kernel,
        out_type=jax.ShapeDtypeStruct((C, DP), jnp.float32),
        mesh=mesh,
        scratch_types=[
            pltpu.VMEM((CW * K,), jnp.int32),
            pltpu.VMEM((RCH, DP), jnp.float32),
            pltpu.VMEM((NCH, DP), jnp.float32),
            pltpu.SemaphoreType.DMA,
        ],
        compiler_params=pltpu.CompilerParams(use_tc_tiling_on_sc=True),
    )
    def body(table_hbm, idx_hbm, out_hbm, idx_v, stg_v, acc_v, sem):
        cid = lax.axis_index("c")
        sid = lax.axis_index("s")
        wid = sid * NC + cid
        base = wid * CW
        pltpu.sync_copy(idx_hbm.at[pl.ds(base * K, CW * K)], idx_v)

        @pl.loop(0, NCHUNK)
        def _chunk(ch):
            roff = pl.multiple_of(ch * RCH, 8)
            pltpu.async_copy(
                table_hbm.at[idx_v.at[pl.ds(roff, RCH)]], stg_v, sem
            ).wait()

            @pl.loop(0, NCH)
            def _concept(i):
                row0 = i * K

                def kbody(k, accs):
                    return tuple(
                        a + stg_v[row0 + k, pl.ds(o, 16)]
                        for a, o in zip(accs, OFFS)
                    )

                accs = tuple(stg_v[row0, pl.ds(o, 16)] for o in OFFS)
                accs = lax.fori_loop(1, K, kbody, accs)
                for a, o in zip(accs, OFFS):
                    acc_v[i, pl.ds(o, 16)] = a

            pltpu.sync_copy(acc_v, out_hbm.at[pl.ds(base + ch * NCH, NCH)])

    return body(table, idx_flat)


def _tc_head(x2, local_h):
    bt = 128

    def body(x_ref, h_ref, o_ref):
        logits = lax.dot_general(
            x_ref[...], h_ref[...], (((1,), (1,)), ((), ())),
            preferred_element_type=jnp.float32,
            precision=lax.Precision.DEFAULT,
        )
        m = jnp.max(logits, axis=1, keepdims=True)
        e = jnp.exp(logits - m)
        o_ref[...] = e / jnp.sum(e, axis=1, keepdims=True)

    return pl.pallas_call(
        body,
        grid=(B // bt,),
        in_specs=[
            pl.BlockSpec((bt, DP), lambda i: (i, 0)),
            pl.BlockSpec((C, DP), lambda i: (0, 0)),
        ],
        out_specs=pl.BlockSpec((bt, C), lambda i: (i, 0)),
        out_shape=jax.ShapeDtypeStruct((B, C), jnp.float32),
    )(x2, local_h)


def kernel(x, y, embed_weight, anc_idx):
    idx_flat = anc_idx.reshape(-1).astype(jnp.int32)
    tpad = jnp.pad(embed_weight, ((0, 0), (0, DP - D)))
    xpad = jnp.pad(x.reshape(B, D), ((0, 0), (0, DP - D)))
    local_h = _sc_gather_sum(tpad, idx_flat)
    return _tc_head(xpad, local_h)
